# baseline (speedup 1.0000x reference)
import functools

import jax
import jax.numpy as jnp
from jax.experimental import pallas as pl
from jax.experimental.pallas import tpu as pltpu

D_MODEL = 512
DF = 128
LN_EPS = 1e-5

BB = 8
TB = 512


def _detector_kernel(x_ref, vbd_ref, bias_ref, w1_ref, b1_ref, gamma_ref,
                     beta_ref, w2_ref, b2_ref, o_ref,
                     s1e, s1l, s2, cnt, mxv, pk, thr_s, *,
                     t_blocks, n_early, t_total):
    phase = pl.program_id(1)
    ti = pl.program_id(2)

    xb3 = x_ref[...]
    x_flat = xb3.reshape(BB * TB, D_MODEL)
    vbd = vbd_ref[...].reshape(BB, BB * TB)

    @pl.when(phase == 0)
    def _accumulate():
        @pl.when(ti == 0)
        def _init():
            s1e[...] = jnp.zeros_like(s1e)
            s1l[...] = jnp.zeros_like(s1l)
            s2[...] = jnp.zeros_like(s2)
            cnt[...] = jnp.zeros_like(cnt)
            mxv[...] = jnp.full_like(mxv, -2e6)

        s1_blk = jnp.dot(vbd, x_flat, preferred_element_type=jnp.float32)

        @pl.when(ti < n_early)
        def _():
            s1e[...] += s1_blk

        @pl.when(ti >= n_early)
        def _():
            s1l[...] += s1_blk

        s2[...] += jnp.dot(vbd, x_flat * x_flat,
                           preferred_element_type=jnp.float32)
        cnt[...] += jnp.sum(vbd, axis=1, keepdims=True)
        masked = (x_flat + bias_ref[...]).reshape(BB, TB, D_MODEL)
        mxv[...] = jnp.maximum(mxv[...], jnp.max(masked, axis=1))

    @pl.when(phase == 1)
    def _peaks():
        @pl.when(ti == 0)
        def _init():
            pk[...] = jnp.zeros_like(pk)
            denom = cnt[...] + 1e-8
            pooled = (s1e[...] + s1l[...]) / denom
            thr_s[...] = jnp.max(pooled, axis=-1, keepdims=True) * 0.7

        thr3 = thr_s[...][:, :, None]
        high = jnp.where(xb3 > thr3, 1.0, 0.0).reshape(BB * TB, D_MODEL)
        pk[...] += jnp.dot(vbd, high, preferred_element_type=jnp.float32)

        @pl.when(ti == t_blocks - 1)
        def _head():
            cntv = cnt[...]
            denom = cntv + 1e-8
            s1 = s1e[...] + s1l[...]
            pooled = s1 / denom
            m_raw = jnp.max(mxv[...], axis=-1, keepdims=True)
            max_strength = jnp.where(cntv < t_total,
                                     jnp.maximum(m_raw, -65000.0), m_raw)
            x_var = (s2[...] - 2.0 * pooled * s1
                     + pooled * pooled * cntv) / denom
            variance = jnp.max(x_var, axis=-1, keepdims=True)
            peak_count = jnp.max(pk[...], axis=-1, keepdims=True)
            early = jnp.max(s1e[...], axis=-1, keepdims=True)
            late = jnp.max(s1l[...], axis=-1, keepdims=True)
            asymmetry = jnp.abs(early - late)
            features = jnp.concatenate(
                [max_strength, variance, peak_count, asymmetry], axis=-1)
            h = jnp.dot(features, w1_ref[...],
                        preferred_element_type=jnp.float32) + b1_ref[...]
            mu = jnp.mean(h, axis=-1, keepdims=True)
            var = jnp.mean((h - mu) ** 2, axis=-1, keepdims=True)
            h = (h - mu) / jnp.sqrt(var + LN_EPS) * gamma_ref[...] + beta_ref[...]
            h = 0.5 * h * (1.0 + jax.lax.erf(h * 0.7071067811865476))
            o_ref[...] = jnp.dot(h, w2_ref[...],
                                 preferred_element_type=jnp.float32) + b2_ref[...]


def kernel(x, padding_mask, W1, b1, gamma, beta, W2, b2):
    B, T, D = x.shape
    t_blocks = T // TB
    b_blocks = B // BB
    n_early = (T // 2) // TB
    valid = 1.0 - padding_mask.astype(jnp.float32)

    v4 = valid.reshape(b_blocks, BB, t_blocks, TB).transpose(0, 2, 1, 3)
    vbd = v4[:, :, :, None, :] * jnp.eye(BB, dtype=jnp.float32)[None, None, :, :, None]
    vbd = vbd.reshape(b_blocks, t_blocks, BB, BB * TB)

    bias = (padding_mask.astype(jnp.float32) * -1e6)
    bias = bias.reshape(b_blocks, BB, t_blocks, TB).transpose(0, 2, 1, 3)
    bias = bias.reshape(b_blocks, t_blocks, BB * TB, 1)

    body = functools.partial(_detector_kernel, t_blocks=t_blocks,
                             n_early=n_early, t_total=float(T))
    out = pl.pallas_call(
        body,
        out_shape=jax.ShapeDtypeStruct((B, DF), jnp.float32),
        grid=(b_blocks, 2, t_blocks),
        in_specs=[
            pl.BlockSpec((BB, TB, D), lambda bi, ph, ti: (bi, ti, 0)),
            pl.BlockSpec((1, 1, BB, BB * TB), lambda bi, ph, ti: (bi, ti, 0, 0)),
            pl.BlockSpec((1, 1, BB * TB, 1), lambda bi, ph, ti: (bi, ti, 0, 0)),
            pl.BlockSpec((4, DF), lambda bi, ph, ti: (0, 0)),
            pl.BlockSpec((1, DF), lambda bi, ph, ti: (0, 0)),
            pl.BlockSpec((1, DF), lambda bi, ph, ti: (0, 0)),
            pl.BlockSpec((1, DF), lambda bi, ph, ti: (0, 0)),
            pl.BlockSpec((DF, DF), lambda bi, ph, ti: (0, 0)),
            pl.BlockSpec((1, DF), lambda bi, ph, ti: (0, 0)),
        ],
        out_specs=pl.BlockSpec((BB, DF), lambda bi, ph, ti: (bi, 0)),
        scratch_shapes=[
            pltpu.VMEM((BB, D), jnp.float32),
            pltpu.VMEM((BB, D), jnp.float32),
            pltpu.VMEM((BB, D), jnp.float32),
            pltpu.VMEM((BB, 1), jnp.float32),
            pltpu.VMEM((BB, D), jnp.float32),
            pltpu.VMEM((BB, D), jnp.float32),
            pltpu.VMEM((BB, 1), jnp.float32),
        ],
        compiler_params=pltpu.CompilerParams(
            dimension_semantics=("parallel", "arbitrary", "arbitrary"),
            vmem_limit_bytes=56 * 1024 * 1024,
        ),
        name="caustic_detector",
    )(x, vbd, bias, W1, b1.reshape(1, DF), gamma.reshape(1, DF),
      beta.reshape(1, DF), W2, b2.reshape(1, DF))
    return out
